# baseline (speedup 1.0000x reference)
import jax
import jax.numpy as jnp
from jax import lax
from jax.experimental import pallas as pl
from jax.experimental.pallas import tpu as pltpu

_E = 8
_H = 1024
_I = 512
_ZC = 0.01
_T = 2048
_IC = 256


def _body(
    x_ref, gw_ref, wg_hbm, wu_hbm, wd_hbm,
    out_ref, logits_ref,
    wg_b, wu_b, wd_b, sem,
):
    def copies(e, p):
        return (
            pltpu.make_async_copy(wg_hbm.at[e], wg_b.at[p], sem.at[p, 0]),
            pltpu.make_async_copy(wu_hbm.at[e], wu_b.at[p], sem.at[p, 1]),
            pltpu.make_async_copy(wd_hbm.at[e], wd_b.at[p], sem.at[p, 2]),
        )

    def start(e, p):
        for c in copies(e, p):
            c.start()

    def wait(e, p):
        for c in copies(e, p):
            c.wait()

    start(0, 0)
    start(1, 1)

    x = x_ref[...]
    logits = lax.dot_general(
        x, gw_ref[...], (((1,), (1,)), ((), ())),
        preferred_element_type=jnp.float32,
    )
    logits_ref[...] = logits
    m = jnp.mean(logits, axis=-1, keepdims=True)
    c = logits - m
    z_loss = _ZC * jnp.sum(c * c) / (_T * _E)
    l2 = logits - z_loss * logits
    rowmax = jnp.max(l2, axis=-1, keepdims=True)
    p = jnp.exp(l2 - rowmax)
    p = p / jnp.sum(p, axis=-1, keepdims=True)
    scores = p / jnp.clip(jnp.sum(p, axis=-1, keepdims=True), 1e-8, None)

    lane = lax.broadcasted_iota(jnp.int32, (1, _E), 1)
    for e in range(_E):
        pb = e % 2
        wait(e, pb)
        s_col = jnp.sum(
            jnp.where(lane == e, scores, 0.0), axis=-1, keepdims=True
        )
        o = None
        for i0 in range(0, _I, _IC):
            g = lax.dot_general(
                x, wg_b[pb, pl.ds(i0, _IC), :], (((1,), (1,)), ((), ())),
                preferred_element_type=jnp.float32,
            )
            u = lax.dot_general(
                x, wu_b[pb, pl.ds(i0, _IC), :], (((1,), (1,)), ((), ())),
                preferred_element_type=jnp.float32,
            )
            h = (g * jax.nn.sigmoid(g) * u) * s_col
            oc = lax.dot_general(
                h, wd_b[pb, :, pl.ds(i0, _IC)], (((1,), (1,)), ((), ())),
                preferred_element_type=jnp.float32,
            )
            o = oc if o is None else o + oc
        if e + 2 < _E:
            start(e + 2, pb)
        if e == 0:
            out_ref[...] = o
        else:
            out_ref[...] += o


@jax.jit
def kernel(hidden_states, gate_w, w_gate, w_up, w_down):
    B, S, H = hidden_states.shape
    T = B * S
    x = hidden_states.reshape(T, H)

    final, router_logits = pl.pallas_call(
        _body,
        in_specs=[
            pl.BlockSpec(memory_space=pltpu.MemorySpace.VMEM),
            pl.BlockSpec(memory_space=pltpu.MemorySpace.VMEM),
            pl.BlockSpec(memory_space=pltpu.MemorySpace.HBM),
            pl.BlockSpec(memory_space=pltpu.MemorySpace.HBM),
            pl.BlockSpec(memory_space=pltpu.MemorySpace.HBM),
        ],
        out_specs=(
            pl.BlockSpec(memory_space=pltpu.MemorySpace.VMEM),
            pl.BlockSpec(memory_space=pltpu.MemorySpace.VMEM),
        ),
        out_shape=(
            jax.ShapeDtypeStruct((T, _H), jnp.float32),
            jax.ShapeDtypeStruct((T, _E), jnp.float32),
        ),
        scratch_shapes=[
            pltpu.VMEM((2, _I, _H), jnp.float32),
            pltpu.VMEM((2, _I, _H), jnp.float32),
            pltpu.VMEM((2, _H, _I), jnp.float32),
            pltpu.SemaphoreType.DMA((2, 3)),
        ],
    )(x, gate_w, w_gate, w_up, w_down)

    return final.reshape(B, S, H), router_logits


# speedup vs baseline: 1.3137x; 1.3137x over previous
import jax
import jax.numpy as jnp
from jax import lax
from jax.experimental import pallas as pl
from jax.experimental.pallas import tpu as pltpu

_E = 8
_H = 1024
_I = 512
_ZC = 0.01
_T = 2048
_IC = 256


def _body(x_ref, gw_ref, wg_ref, wu_ref, wd_ref, out_ref, logits_ref, scores_s):
    e = pl.program_id(0)

    @pl.when(e == 0)
    def _router():
        logits = lax.dot_general(
            x_ref[...], gw_ref[...], (((1,), (1,)), ((), ())),
            preferred_element_type=jnp.float32,
        )
        logits_ref[...] = logits
        m = jnp.mean(logits, axis=-1, keepdims=True)
        c = logits - m
        z_loss = _ZC * jnp.sum(c * c) / (_T * _E)
        l2 = logits - z_loss * logits
        rowmax = jnp.max(l2, axis=-1, keepdims=True)
        p = jnp.exp(l2 - rowmax)
        p = p / jnp.sum(p, axis=-1, keepdims=True)
        scores_s[...] = p / jnp.clip(jnp.sum(p, axis=-1, keepdims=True), 1e-8, None)

    x = x_ref[...]
    lane = lax.broadcasted_iota(jnp.int32, (1, _E), 1)
    s_col = jnp.sum(
        jnp.where(lane == e, scores_s[...], 0.0), axis=-1, keepdims=True
    )

    for i0 in range(0, _I, _IC):
        g = lax.dot_general(
            x, wg_ref[0, pl.ds(i0, _IC), :], (((1,), (1,)), ((), ())),
            preferred_element_type=jnp.float32,
        )
        u = lax.dot_general(
            x, wu_ref[0, pl.ds(i0, _IC), :], (((1,), (1,)), ((), ())),
            preferred_element_type=jnp.float32,
        )
        h = (g * jax.nn.sigmoid(g) * u) * s_col
        oc = lax.dot_general(
            h, wd_ref[0, :, pl.ds(i0, _IC)], (((1,), (1,)), ((), ())),
            preferred_element_type=jnp.float32,
        )
        o = oc if i0 == 0 else o + oc

    @pl.when(e == 0)
    def _():
        out_ref[...] = o

    @pl.when(e != 0)
    def _():
        out_ref[...] += o


@jax.jit
def kernel(hidden_states, gate_w, w_gate, w_up, w_down):
    B, S, H = hidden_states.shape
    T = B * S
    x = hidden_states.reshape(T, H)

    final, router_logits = pl.pallas_call(
        _body,
        grid=(_E,),
        in_specs=[
            pl.BlockSpec((T, _H), lambda e: (0, 0)),
            pl.BlockSpec((_E, _H), lambda e: (0, 0)),
            pl.BlockSpec((1, _I, _H), lambda e: (e, 0, 0)),
            pl.BlockSpec((1, _I, _H), lambda e: (e, 0, 0)),
            pl.BlockSpec((1, _H, _I), lambda e: (e, 0, 0)),
        ],
        out_specs=(
            pl.BlockSpec((T, _H), lambda e: (0, 0)),
            pl.BlockSpec((T, _E), lambda e: (0, 0)),
        ),
        out_shape=(
            jax.ShapeDtypeStruct((T, _H), jnp.float32),
            jax.ShapeDtypeStruct((T, _E), jnp.float32),
        ),
        scratch_shapes=[pltpu.VMEM((T, _E), jnp.float32)],
        compiler_params=pltpu.CompilerParams(
            dimension_semantics=("arbitrary",),
        ),
    )(x, gate_w, w_gate, w_up, w_down)

    return final.reshape(B, S, H), router_logits
